# Initial kernel scaffold; baseline (speedup 1.0000x reference)
import jax
import jax.numpy as jnp
from jax.experimental import pallas as pl


def kernel(h, e, edge_index_full, adj2, rel_pos_3d, Wq, bq, Wk, bk, Wv, bv, Wpe, bpe, Wap, bap, Wo, bo, Woe, boe):
    raise NotImplementedError("write your pallas kernel here")


# speedup vs baseline: 8.8181x; 8.8181x over previous
import functools

import jax
import jax.numpy as jnp
import numpy as np
from jax import lax
from jax.experimental import pallas as pl
from jax.experimental.pallas import tpu as pltpu
from jax.experimental.pallas import tpu_sc as plsc

N = 10000
IN_DIM = 128
H = 8
D = 16
ED = 16
EG = 160000
EF = 320000
HD = H * D

NC = 2
NS = 16
L = 16

NWORK = NS
EW = EF // NWORK
EB = 80
NB = EW // EB
NG = EB * H // L
NPAD = 10240
ROWS_PER_SUB = NPAD // NS
ZCHUNK = 80
ZZLEN = 1024

_INV_SQRT_D = float(1.0 / np.sqrt(D))


def _qkv_body(h_ref, wqt, wkt, wvt, bq, bk, bv, q_ref, k_ref, v_ref):
    hb = h_ref[...]
    q_ref[...] = jnp.dot(hb, wqt[...], preferred_element_type=jnp.float32) + bq[...]
    k_ref[...] = jnp.dot(hb, wkt[...], preferred_element_type=jnp.float32) + bk[...]
    v_ref[...] = jnp.dot(hb, wvt[...], preferred_element_type=jnp.float32) + bv[...]


def _proj_e_body(e_ref, wpet, bpe, out_ref):
    out_ref[...] = jnp.dot(e_ref[...], wpet[...],
                           preferred_element_type=jnp.float32) + bpe[...]


def _hout_body(wv0, z0, zexp_m, wot, bo, out_ref):
    wv = wv0[...]
    z = z0[...]
    zex = jnp.dot(z, zexp_m[...], preferred_element_type=jnp.float32)
    t = wv / (zex + 1e-6)
    out_ref[...] = jnp.dot(t, wot[...], preferred_element_type=jnp.float32) + bo[...]


def _eout_body(s1_ref, e_ref, wapt, bap, woet, boe, out_ref):
    ep = jnp.dot(s1_ref[...], wapt[...], preferred_element_type=jnp.float32) + bap[...]
    out_ref[...] = jnp.dot(ep + e_ref[...], woet[...],
                           preferred_element_type=jnp.float32) + boe[...]


def _sc_body(q_hbm, k_hbm, v_hbm, src_hbm, dst_hbm, adj_hbm, rel_hbm, proj_hbm,
             s1_out, wv_out, z_out,
             idxs_v, idxd_v, rowsK, rowsQ, rowsV,
             sc1_v, rel_v, proj_v, adj_v, finT,
             zidx0, zidx1, zidx2, zidx3, zidx4, zidx5, zidx6, zidx7,
             zz_v, wv_sh, z_sh,
             semK, semQ, semV):
    zidxs = (zidx0, zidx1, zidx2, zidx3, zidx4, zidx5, zidx6, zidx7)
    sid = lax.axis_index("s")

    zeros16 = jnp.zeros((L,), jnp.float32)

    def _zero_proj(i, _):
        proj_v[pl.ds(i * L, L)] = zeros16
        return 0
    lax.fori_loop(0, EB * H // L, _zero_proj, 0)

    def _zero_zz(i, _):
        zz_v[pl.ds(i * L, L)] = zeros16
        return 0
    lax.fori_loop(0, ZZLEN // L, _zero_zz, 0)

    def _zero_rows(i, _):
        for j in range(HD // L):
            rowsK[i, pl.ds(j * L, L)] = zeros16
        return 0
    lax.fori_loop(0, EB, _zero_rows, 0)

    row0 = sid * ROWS_PER_SUB
    for t in range(ROWS_PER_SUB // ZCHUNK):
        pltpu.sync_copy(rowsK, wv_sh.at[pl.ds(row0 + t * ZCHUNK, ZCHUNK), :])
    zoff0 = row0 * L
    for t in range(ROWS_PER_SUB * L // ZZLEN):
        pltpu.sync_copy(zz_v, z_sh.at[pl.ds(zoff0 + t * ZZLEN, ZZLEN)])
    plsc.subcore_barrier()

    iota = lax.iota(jnp.int32, L)
    lane_row = lax.shift_right_logical(iota, 3)
    lane_head = lax.bitwise_and(iota, 7)
    lane_col0 = lane_head * D

    wbase = sid * EW

    def _block(b, _):
        base = pl.multiple_of(wbase + b * EB, EB)
        base8 = pl.multiple_of(base * H, EB * H)

        pltpu.sync_copy(src_hbm.at[pl.ds(base, EB)], idxs_v)
        pltpu.sync_copy(dst_hbm.at[pl.ds(base, EB)], idxd_v)
        pltpu.sync_copy(adj_hbm.at[pl.ds(base, EB)], adj_v)
        pltpu.sync_copy(rel_hbm.at[pl.ds(base8, EB * H)], rel_v)

        @pl.when(base < EG)
        def _():
            pltpu.sync_copy(proj_hbm.at[pl.ds(base8, EB * H)], proj_v)

        cK = pltpu.async_copy(k_hbm.at[idxs_v], rowsK, semK)
        cQ = pltpu.async_copy(q_hbm.at[idxd_v], rowsQ, semQ)
        cV = pltpu.async_copy(v_hbm.at[idxs_v], rowsV, semV)
        cK.wait()
        cQ.wait()
        cV.wait()

        def _group(g, _):
            ridx = lane_row + 2 * g
            acc = jnp.zeros((L,), jnp.float32)
            for d in range(D):
                cidx = lane_col0 + d
                kv = plsc.load_gather(rowsK, [ridx, cidx])
                qv = plsc.load_gather(rowsQ, [ridx, cidx])
                acc = acc + kv * qv
            s = acc * _INV_SQRT_D
            a = plsc.load_gather(adj_v, [ridx])
            goff = pl.multiple_of(g * L, L)
            sc1 = jnp.exp(jnp.clip(s, -5.0, 5.0) * a) + rel_v[pl.ds(goff, L)]
            sc1_v[pl.ds(goff, L)] = sc1
            fin = jnp.exp(jnp.clip(jnp.clip(sc1, -5.0, 5.0)
                                   + proj_v[pl.ds(goff, L)], -5.0, 5.0))
            plsc.store_scatter(finT, [lane_head, ridx], fin)
            for d in range(D):
                cidx = lane_col0 + d
                vv = plsc.load_gather(rowsV, [ridx, cidx])
                plsc.store_scatter(rowsV, [ridx, cidx], vv * fin)
            return 0
        lax.fori_loop(0, NG, _group, 0)

        pltpu.sync_copy(sc1_v, s1_out.at[pl.ds(base8, EB * H)])

        for c in range(EB // L):
            dv = idxd_v[pl.ds(c * L, L)] * L
            for j in range(H):
                zidxs[j][pl.ds(c * L, L)] = dv + j

        pltpu.sync_copy(rowsV, wv_sh.at[idxd_v], add=True)
        for j in range(H):
            pltpu.sync_copy(finT.at[j, :], z_sh.at[zidxs[j]], add=True)
        return 0

    lax.fori_loop(0, NB, _block, 0)

    plsc.subcore_barrier()

    for t in range(ROWS_PER_SUB // ZCHUNK):
        r = row0 + t * ZCHUNK
        pltpu.sync_copy(wv_sh.at[pl.ds(r, ZCHUNK), :],
                        wv_out.at[pl.ds(r, ZCHUNK), :])
    for t in range(ROWS_PER_SUB * L // ZZLEN):
        o = zoff0 + t * ZZLEN
        pltpu.sync_copy(z_sh.at[pl.ds(o, ZZLEN)], z_out.at[pl.ds(o, ZZLEN)])


def _sc_call(q, k, v, src, dst, adj2, rel2, proj2):
    mesh = plsc.VectorSubcoreMesh(core_axis_name="c", subcore_axis_name="s",
                                  num_cores=1, num_subcores=NS)
    out_type = (
        jax.ShapeDtypeStruct((EF * H,), jnp.float32),
        jax.ShapeDtypeStruct((NPAD, HD), jnp.float32),
        jax.ShapeDtypeStruct((NPAD * L,), jnp.float32),
    )
    scratch = [
        pltpu.VMEM((EB,), jnp.int32),
        pltpu.VMEM((EB,), jnp.int32),
        pltpu.VMEM((EB, HD), jnp.float32),
        pltpu.VMEM((EB, HD), jnp.float32),
        pltpu.VMEM((EB, HD), jnp.float32),
        pltpu.VMEM((EB * H,), jnp.float32),
        pltpu.VMEM((EB * H,), jnp.float32),
        pltpu.VMEM((EB * H,), jnp.float32),
        pltpu.VMEM((EB,), jnp.float32),
        pltpu.VMEM((H, EB), jnp.float32),
    ] + [pltpu.VMEM((EB,), jnp.int32)] * H + [
        pltpu.VMEM((ZZLEN,), jnp.float32),
        pltpu.VMEM_SHARED((NPAD, HD), jnp.float32),
        pltpu.VMEM_SHARED((NPAD * L,), jnp.float32),
        pltpu.SemaphoreType.DMA,
        pltpu.SemaphoreType.DMA,
        pltpu.SemaphoreType.DMA,
    ]
    kern = pl.kernel(_sc_body, out_type=out_type, mesh=mesh,
                     scratch_types=scratch,
                     compiler_params=pltpu.CompilerParams(
                         needs_layout_passes=False))
    return kern(q, k, v, src, dst, adj2, rel2, proj2)


_NROW = 400
_EROW = 1000


@jax.jit
def kernel(h, e, edge_index_full, adj2, rel_pos_3d, Wq, bq, Wk, bk, Wv, bv,
           Wpe, bpe, Wap, bap, Wo, bo, Woe, boe):
    src = edge_index_full[0].astype(jnp.int32)
    dst = edge_index_full[1].astype(jnp.int32)

    qkv = pl.pallas_call(
        _qkv_body,
        grid=(N // _NROW,),
        in_specs=[
            pl.BlockSpec((_NROW, IN_DIM), lambda i: (i, 0)),
            pl.BlockSpec((IN_DIM, HD), lambda i: (0, 0)),
            pl.BlockSpec((IN_DIM, HD), lambda i: (0, 0)),
            pl.BlockSpec((IN_DIM, HD), lambda i: (0, 0)),
            pl.BlockSpec((1, HD), lambda i: (0, 0)),
            pl.BlockSpec((1, HD), lambda i: (0, 0)),
            pl.BlockSpec((1, HD), lambda i: (0, 0)),
        ],
        out_specs=[
            pl.BlockSpec((_NROW, HD), lambda i: (i, 0)),
            pl.BlockSpec((_NROW, HD), lambda i: (i, 0)),
            pl.BlockSpec((_NROW, HD), lambda i: (i, 0)),
        ],
        out_shape=[jax.ShapeDtypeStruct((N, HD), jnp.float32)] * 3,
    )(h, Wq.T, Wk.T, Wv.T, bq.reshape(1, HD), bk.reshape(1, HD),
      bv.reshape(1, HD))
    q, k, v = qkv

    proj_e = pl.pallas_call(
        _proj_e_body,
        grid=(EG // _EROW,),
        in_specs=[
            pl.BlockSpec((_EROW, ED), lambda i: (i, 0)),
            pl.BlockSpec((ED, H), lambda i: (0, 0)),
            pl.BlockSpec((1, H), lambda i: (0, 0)),
        ],
        out_specs=pl.BlockSpec((_EROW, H), lambda i: (i, 0)),
        out_shape=jax.ShapeDtypeStruct((EG, H), jnp.float32),
    )(e, Wpe.T, bpe.reshape(1, H))

    rel2 = rel_pos_3d.reshape(EF * H)
    proj2 = proj_e.reshape(EG * H)

    s1_2, wv_acc, z_flat = _sc_call(q, k, v, src, dst, adj2, rel2, proj2)
    z_acc = z_flat.reshape(NPAD, L)
    score1_g = s1_2.reshape(EF, H)[:EG]


    zexp_m = np.zeros((L, HD), np.float32)
    for hh in range(H):
        zexp_m[hh, hh * D:(hh + 1) * D] = 1.0
    zexp_m = jnp.asarray(zexp_m)

    h_out = pl.pallas_call(
        _hout_body,
        grid=(N // _NROW,),
        in_specs=[
            pl.BlockSpec((_NROW, HD), lambda i: (i, 0)),
            pl.BlockSpec((_NROW, L), lambda i: (i, 0)),
            pl.BlockSpec((L, HD), lambda i: (0, 0)),
            pl.BlockSpec((HD, HD), lambda i: (0, 0)),
            pl.BlockSpec((1, HD), lambda i: (0, 0)),
        ],
        out_specs=pl.BlockSpec((_NROW, HD), lambda i: (i, 0)),
        out_shape=jax.ShapeDtypeStruct((N, HD), jnp.float32),
    )(wv_acc, z_acc, zexp_m, Wo.T, bo.reshape(1, HD))

    e_out = pl.pallas_call(
        _eout_body,
        grid=(EG // _EROW,),
        in_specs=[
            pl.BlockSpec((_EROW, H), lambda i: (i, 0)),
            pl.BlockSpec((_EROW, ED), lambda i: (i, 0)),
            pl.BlockSpec((H, ED), lambda i: (0, 0)),
            pl.BlockSpec((1, ED), lambda i: (0, 0)),
            pl.BlockSpec((ED, ED), lambda i: (0, 0)),
            pl.BlockSpec((1, ED), lambda i: (0, 0)),
        ],
        out_specs=pl.BlockSpec((_EROW, ED), lambda i: (i, 0)),
        out_shape=jax.ShapeDtypeStruct((EG, ED), jnp.float32),
    )(score1_g, e, Wap.T, bap.reshape(1, ED), Woe.T, boe.reshape(1, ED))

    return (h_out, e_out)
